# Initial kernel scaffold; baseline (speedup 1.0000x reference)
import jax
import jax.numpy as jnp
from jax.experimental import pallas as pl


def kernel(x, edge_index, lin_w, lin_b, c1_w1, c1_b1, c1_w2, c1_b2, c1_g, c1_bb, bn1_g, bn1_b, c2_w1, c2_b1, c2_w2, c2_b2, c2_g, c2_bb, bn2_g, bn2_b, c3_w1, c3_b1, c3_w2, c3_b2, c3_g, c3_bb, bn3_g, bn3_b):
    raise NotImplementedError("write your pallas kernel here")


# speedup vs baseline: 1.1425x; 1.1425x over previous
import functools

import jax
import jax.numpy as jnp
from jax import lax
from jax.experimental import pallas as pl
from jax.experimental.pallas import tpu as pltpu
from jax.experimental.pallas import tpu_sc as plsc

F32 = jnp.float32
I32 = jnp.int32
EPSV = 1e-05
NCORE, NSUB = 2, 16
NW = NCORE * NSUB
CG = 120
CS = 128
SEC = 4
SFG = 8
BLK = 512
HP = 128
NEG = -3.0e38
POS = 3.0e38


def _lin(x, lin_w, lin_b):
    n = x.shape[0]
    o = lin_w.shape[1]

    def body(x_r, lw_r, lb_r, h_r):
        h0 = jnp.dot(x_r[...], lw_r[...], preferred_element_type=F32) + lb_r[...]
        h_r[...] = jnp.concatenate([h0, jnp.zeros((n, HP - o), F32)], axis=1)

    return pl.pallas_call(
        body,
        out_shape=jax.ShapeDtypeStruct((n, HP), F32),
    )(x, lin_w, lin_b.reshape(1, o))


def _sc_gather(hp, srcp, dstp, fin):
    e2p = srcp.shape[0]
    fm = 2 * fin
    nepw = e2p // NW
    nch = nepw // CG
    mesh = plsc.VectorSubcoreMesh(core_axis_name="c", subcore_axis_name="s")

    @functools.partial(
        pl.kernel,
        out_type=jax.ShapeDtypeStruct((e2p, fm), F32),
        mesh=mesh,
        scratch_types=[
            pltpu.VMEM((CG,), I32),
            pltpu.VMEM((CG,), I32),
            pltpu.VMEM((CG, HP), F32),
            pltpu.VMEM((CG, HP), F32),
            pltpu.VMEM((CG, fm), F32),
            pltpu.SemaphoreType.DMA,
        ],
    )
    def k(h_hbm, s_hbm, d_hbm, out_hbm, dbuf, sbuf, dr, sr, ob, sem):
        wid = lax.axis_index("s") * NCORE + lax.axis_index("c")
        base = wid * nepw

        def chunk(ci, carry):
            e0 = base + ci * CG
            pltpu.sync_copy(d_hbm.at[pl.ds(e0, CG)], dbuf)
            pltpu.sync_copy(s_hbm.at[pl.ds(e0, CG)], sbuf)
            ca = pltpu.async_copy(h_hbm.at[dbuf], dr, sem)
            cb = pltpu.async_copy(h_hbm.at[sbuf], sr, sem)
            ca.wait()
            cb.wait()

            def row(i, c2):
                for j in range(fin // 16):
                    sl = pl.ds(j * 16, 16)
                    sl2 = pl.ds(fin + j * 16, 16)
                    xi = dr[i, sl]
                    ob[i, sl] = xi
                    ob[i, sl2] = sr[i, sl] - xi
                return c2

            lax.fori_loop(0, CG, row, 0)
            pltpu.sync_copy(ob, out_hbm.at[pl.ds(e0, CG)])
            return carry

        lax.fori_loop(0, nch, chunk, 0)

    return k(hp, srcp, dstp)


def _edge_mlp(m, w1, b1, w2, b2, e2):
    e2p, fm = m.shape
    h = w1.shape[1]
    fo = w2.shape[1]
    nb = e2p // BLK

    def body(m_r, w1_r, b1_r, w2_r, b2_r, h2_r, st_r):
        i = pl.program_id(0)
        h1 = jnp.maximum(
            jnp.dot(m_r[...], w1_r[...], preferred_element_type=F32) + b1_r[...],
            0.0)
        hh = jnp.maximum(
            jnp.dot(h1, w2_r[...], preferred_element_type=F32) + b2_r[...], 0.0)
        h2_r[...] = hh
        rows = i * BLK + lax.broadcasted_iota(I32, (BLK, 1), 0)
        hm = jnp.where(rows < e2, hh, 0.0)
        ps = jnp.sum(hm, axis=0, keepdims=True)
        pq = jnp.sum(hm * hm, axis=0, keepdims=True)

        @pl.when(i == 0)
        def _():
            st_r[...] = jnp.zeros_like(st_r)

        st_r[0:1, :] = st_r[0:1, :] + ps
        st_r[1:2, :] = st_r[1:2, :] + pq

    return pl.pallas_call(
        body,
        grid=(nb,),
        in_specs=[
            pl.BlockSpec((BLK, fm), lambda i: (i, 0)),
            pl.BlockSpec((fm, h), lambda i: (0, 0)),
            pl.BlockSpec((1, h), lambda i: (0, 0)),
            pl.BlockSpec((h, fo), lambda i: (0, 0)),
            pl.BlockSpec((1, fo), lambda i: (0, 0)),
        ],
        out_specs=[
            pl.BlockSpec((BLK, fo), lambda i: (i, 0)),
            pl.BlockSpec((8, fo), lambda i: (0, 0)),
        ],
        out_shape=(jax.ShapeDtypeStruct((e2p, fo), F32),
                   jax.ShapeDtypeStruct((8, fo), F32)),
    )(m, w1, b1.reshape(1, h), w2, b2.reshape(1, fo))


def _sc_segext(h2, dstp, n, is_max):
    e2p, fo = h2.shape
    epw = e2p // SEC
    nch = epw // CS
    nflat = n * 8
    msize = nflat + 16
    fill = NEG if is_max else POS
    mesh = plsc.VectorSubcoreMesh(core_axis_name="c", subcore_axis_name="s")

    @functools.partial(
        pl.kernel,
        out_type=jax.ShapeDtypeStruct((SEC, SFG, msize), F32),
        mesh=mesh,
        scratch_types=[
            pltpu.VMEM((msize,), F32),
            pltpu.VMEM((CS,), I32),
            pltpu.VMEM((CS, 64), F32),
        ],
    )
    def k(h2_hbm, d_hbm, out_hbm, mbuf, dbuf, vbuf):
        wid = lax.axis_index("s") * NCORE + lax.axis_index("c")
        ec = wid // SFG
        fg = wid - ec * SFG
        off = jnp.minimum(fg * 8, fo - 16)
        own0 = fg * 8 - off
        sbias = 8 - own0
        iota = lax.broadcasted_iota(I32, (16,), 0)
        ownmask = (iota >= own0) & (iota < own0 + 8)
        init = jnp.full((16,), fill, F32)

        def initrow(i, c):
            mbuf[pl.ds(i * 16, 16)] = init
            return c

        lax.fori_loop(0, msize // 16, initrow, 0)
        base = ec * epw

        def chunk(ci, c):
            e0 = base + ci * CS
            pltpu.sync_copy(d_hbm.at[pl.ds(e0, CS)], dbuf)
            pltpu.sync_copy(h2_hbm.at[pl.ds(e0, CS)], vbuf)

            def grp(gi, c2):
                dvec = dbuf[pl.ds(gi * 16, 16)]
                for j in range(16):
                    dd = dvec[j]
                    s0 = dd * 8 + sbias
                    vals = vbuf[gi * 16 + j, pl.ds(off, 16)]
                    cur = mbuf[pl.ds(s0, 16)]
                    nv = (jnp.maximum(cur, vals) if is_max
                          else jnp.minimum(cur, vals))
                    mbuf[pl.ds(s0, 16)] = jnp.where(ownmask, nv, cur)
                return c2

            lax.fori_loop(0, CS // 16, grp, 0)
            return c

        lax.fori_loop(0, nch, chunk, 0)
        pltpu.sync_copy(mbuf, out_hbm.at[ec, fg])

    return k(h2, dstp)


def _node(maxp, minp, stats, g, bb, ng, nb_, relu, e2, pad_out):
    _, n, fo = maxp.shape
    fout = HP if pad_out else fo

    def body(mx_r, mn_r, st_r, g_r, bb_r, ng_r, nb_r, out_r):
        m_hi = jnp.max(mx_r[...], axis=0)
        m_lo = jnp.min(mn_r[...], axis=0)
        s = st_r[0:1, :]
        ss = st_r[1:2, :]
        mu = s / e2
        var = ss / e2 - mu * mu
        sc = g_r[...] / jnp.sqrt(var + EPSV)
        t = bb_r[...] - mu * sc
        y = jnp.where(sc >= 0, m_hi * sc, m_lo * sc) + t
        mun = jnp.mean(y, axis=0, keepdims=True)
        varn = jnp.mean((y - mun) ** 2, axis=0, keepdims=True)
        z = (y - mun) / jnp.sqrt(varn + EPSV) * ng_r[...] + nb_r[...]
        if relu:
            z = jnp.maximum(z, 0.0)
        if pad_out:
            z = jnp.concatenate([z, jnp.zeros((n, HP - fo), F32)], axis=1)
        out_r[...] = z

    return pl.pallas_call(
        body,
        out_shape=jax.ShapeDtypeStruct((n, fout), F32),
    )(maxp, minp, stats, g.reshape(1, fo), bb.reshape(1, fo),
      ng.reshape(1, fo), nb_.reshape(1, fo))


def kernel(x, edge_index, lin_w, lin_b,
           c1_w1, c1_b1, c1_w2, c1_b2, c1_g, c1_bb, bn1_g, bn1_b,
           c2_w1, c2_b1, c2_w2, c2_b2, c2_g, c2_bb, bn2_g, bn2_b,
           c3_w1, c3_b1, c3_w2, c3_b2, c3_g, c3_bb, bn3_g, bn3_b):
    n = x.shape[0]
    o = lin_w.shape[1]
    fo = 2 * o
    e = edge_index.shape[1]
    e2 = e + n
    quant = 7680
    e2p = -(-e2 // quant) * quant
    pad = e2p - e2

    ei = edge_index.astype(I32)
    loops = jnp.arange(n, dtype=I32)
    zpad = jnp.zeros((pad,), I32)
    srcp = jnp.concatenate([ei[0], loops, zpad])
    dstp = jnp.concatenate([ei[1], loops, zpad])

    layers = [
        (c1_w1, c1_b1, c1_w2, c1_b2, c1_g, c1_bb, bn1_g, bn1_b, True),
        (c2_w1, c2_b1, c2_w2, c2_b2, c2_g, c2_bb, bn2_g, bn2_b, True),
        (c3_w1, c3_b1, c3_w2, c3_b2, c3_g, c3_bb, bn3_g, bn3_b, False),
    ]
    hp = _lin(x, lin_w, lin_b)
    fin = o
    for li, (w1, b1, w2, b2, g, bb, ng, nbb, relu) in enumerate(layers):
        m = _sc_gather(hp, srcp, dstp, fin)
        h2, stats = _edge_mlp(m, w1, b1, w2, b2, e2)
        mx = _sc_segext(h2, dstp, n, True)
        mn = lax.cond(
            jnp.any(g < 0),
            lambda hh, dd: _sc_segext(hh, dd, n, False),
            lambda hh, dd: jnp.zeros((SEC, SFG, n * 8 + 16), F32),
            h2, dstp)
        mxr = (mx[:, :, 8:8 + n * 8].reshape(SEC, SFG, n, 8)
               .transpose(0, 2, 1, 3).reshape(SEC, n, fo))
        mnr = (mn[:, :, 8:8 + n * 8].reshape(SEC, SFG, n, 8)
               .transpose(0, 2, 1, 3).reshape(SEC, n, fo))
        hp = _node(mxr, mnr, stats, g, bb, ng, nbb, relu, e2,
                   pad_out=(li < 2))
        fin = fo
    return hp


# speedup vs baseline: 1.8549x; 1.6236x over previous
import functools

import jax
import jax.numpy as jnp
from jax import lax
from jax.experimental import pallas as pl
from jax.experimental.pallas import tpu as pltpu
from jax.experimental.pallas import tpu_sc as plsc

F32 = jnp.float32
I32 = jnp.int32
EPSV = 1e-05
NCORE, NSUB = 2, 16
NW = NCORE * NSUB
CG = 120
CS = 160
SEC = 4
SFG = 8
BLK = 512
HP = 128
NEG = -3.0e38
POS = 3.0e38


def _lin(x, lin_w, lin_b):
    n = x.shape[0]
    o = lin_w.shape[1]

    def body(x_r, lw_r, lb_r, h_r):
        h0 = jnp.dot(x_r[...], lw_r[...], preferred_element_type=F32) + lb_r[...]
        h_r[...] = jnp.concatenate([h0, jnp.zeros((n, HP - o), F32)], axis=1)

    return pl.pallas_call(
        body,
        out_shape=jax.ShapeDtypeStruct((n, HP), F32),
    )(x, lin_w, lin_b.reshape(1, o))


def _sc_gather(hp, srcp, dstp, fin):
    e2p = srcp.shape[0]
    fm = 2 * fin
    nepw = e2p // NW
    nch = nepw // CG
    mesh = plsc.VectorSubcoreMesh(core_axis_name="c", subcore_axis_name="s")

    @functools.partial(
        pl.kernel,
        out_type=jax.ShapeDtypeStruct((e2p, fm), F32),
        mesh=mesh,
        scratch_types=[
            pltpu.VMEM((CG,), I32),
            pltpu.VMEM((CG,), I32),
            pltpu.VMEM((CG,), I32),
            pltpu.VMEM((CG,), I32),
            pltpu.VMEM((CG, HP), F32),
            pltpu.VMEM((CG, HP), F32),
            pltpu.VMEM((CG, HP), F32),
            pltpu.VMEM((CG, HP), F32),
            pltpu.VMEM((CG, fm), F32),
            pltpu.VMEM((CG, fm), F32),
            pltpu.SemaphoreType.DMA,
            pltpu.SemaphoreType.DMA,
            pltpu.SemaphoreType.DMA,
            pltpu.SemaphoreType.DMA,
        ],
    )
    def k(h_hbm, s_hbm, d_hbm, out_hbm,
          dbuf0, dbuf1, sbuf0, sbuf1, dr0, dr1, sr0, sr1, ob0, ob1,
          semg0, semg1, semo0, semo1):
        wid = lax.axis_index("s") * NCORE + lax.axis_index("c")
        base = wid * nepw
        db = (dbuf0, dbuf1)
        sb = (sbuf0, sbuf1)
        dr = (dr0, dr1)
        sr = (sr0, sr1)
        ob = (ob0, ob1)
        semg = (semg0, semg1)
        semo = (semo0, semo1)

        def issue(ci, nb):
            e0 = base + ci * CG
            pltpu.sync_copy(d_hbm.at[pl.ds(e0, CG)], db[nb])
            pltpu.sync_copy(s_hbm.at[pl.ds(e0, CG)], sb[nb])
            pltpu.async_copy(h_hbm.at[db[nb]], dr[nb], semg[nb])
            pltpu.async_copy(h_hbm.at[sb[nb]], sr[nb], semg[nb])

        issue(0, 0)

        def pairbody(gi, carry):
            for b in range(2):
                ci = gi * 2 + b
                nxt = lax.rem(ci + 1, nch)
                issue(nxt, 1 - b)
                pltpu.make_async_copy(h_hbm.at[db[b]], dr[b], semg[b]).wait()
                pltpu.make_async_copy(h_hbm.at[sb[b]], sr[b], semg[b]).wait()

                @pl.when(ci >= 2)
                def _():
                    pltpu.make_async_copy(
                        ob[b], out_hbm.at[pl.ds(base, CG)], semo[b]).wait()

                def row(i, c2):
                    for j in range(fin // 16):
                        sl = pl.ds(j * 16, 16)
                        sl2 = pl.ds(fin + j * 16, 16)
                        xi = dr[b][i, sl]
                        ob[b][i, sl] = xi
                        ob[b][i, sl2] = sr[b][i, sl] - xi
                    return c2

                lax.fori_loop(0, CG, row, 0)
                e0 = base + ci * CG
                pltpu.async_copy(ob[b], out_hbm.at[pl.ds(e0, CG)], semo[b])
            return carry

        lax.fori_loop(0, nch // 2, pairbody, 0)
        pltpu.make_async_copy(h_hbm.at[db[0]], dr[0], semg[0]).wait()
        pltpu.make_async_copy(h_hbm.at[sb[0]], sr[0], semg[0]).wait()
        for b in range(2):
            pltpu.make_async_copy(
                ob[b], out_hbm.at[pl.ds(base, CG)], semo[b]).wait()

    return k(hp, srcp, dstp)


def _edge_mlp(m, w1, b1, w2, b2, e2):
    e2p, fm = m.shape
    h = w1.shape[1]
    fo = w2.shape[1]
    nb = e2p // BLK

    def body(m_r, w1_r, b1_r, w2_r, b2_r, h2_r, st_r):
        i = pl.program_id(0)
        h1 = jnp.maximum(
            jnp.dot(m_r[...], w1_r[...], preferred_element_type=F32) + b1_r[...],
            0.0)
        hh = jnp.maximum(
            jnp.dot(h1, w2_r[...], preferred_element_type=F32) + b2_r[...], 0.0)
        h2_r[...] = hh
        rows = i * BLK + lax.broadcasted_iota(I32, (BLK, 1), 0)
        hm = jnp.where(rows < e2, hh, 0.0)
        ps = jnp.sum(hm, axis=0, keepdims=True)
        pq = jnp.sum(hm * hm, axis=0, keepdims=True)

        @pl.when(i == 0)
        def _():
            st_r[...] = jnp.zeros_like(st_r)

        st_r[0:1, :] = st_r[0:1, :] + ps
        st_r[1:2, :] = st_r[1:2, :] + pq

    return pl.pallas_call(
        body,
        grid=(nb,),
        in_specs=[
            pl.BlockSpec((BLK, fm), lambda i: (i, 0)),
            pl.BlockSpec((fm, h), lambda i: (0, 0)),
            pl.BlockSpec((1, h), lambda i: (0, 0)),
            pl.BlockSpec((h, fo), lambda i: (0, 0)),
            pl.BlockSpec((1, fo), lambda i: (0, 0)),
        ],
        out_specs=[
            pl.BlockSpec((BLK, fo), lambda i: (i, 0)),
            pl.BlockSpec((8, fo), lambda i: (0, 0)),
        ],
        out_shape=(jax.ShapeDtypeStruct((e2p, fo), F32),
                   jax.ShapeDtypeStruct((8, fo), F32)),
    )(m, w1, b1.reshape(1, h), w2, b2.reshape(1, fo))


def _sc_segext(h2, dstp, n, is_max):
    e2p, fo = h2.shape
    epw = e2p // SEC
    nch = epw // CS
    nflat = n * 8
    msize = nflat + 16
    fill = NEG if is_max else POS
    mesh = plsc.VectorSubcoreMesh(core_axis_name="c", subcore_axis_name="s")

    @functools.partial(
        pl.kernel,
        out_type=jax.ShapeDtypeStruct((SEC, SFG, msize), F32),
        mesh=mesh,
        scratch_types=[
            pltpu.VMEM((msize,), F32),
            pltpu.VMEM((CS,), I32),
            pltpu.VMEM((CS,), I32),
            pltpu.VMEM((CS, 64), F32),
            pltpu.VMEM((CS, 64), F32),
            pltpu.SemaphoreType.DMA,
            pltpu.SemaphoreType.DMA,
        ],
    )
    def k(h2_hbm, d_hbm, out_hbm, mbuf, dbuf0, dbuf1, vbuf0, vbuf1,
          sem0, sem1):
        wid = lax.axis_index("s") * NCORE + lax.axis_index("c")
        ec = wid // SFG
        fg = wid - ec * SFG
        off = jnp.minimum(fg * 8, fo - 16)
        own0 = fg * 8 - off
        sbias = 8 - own0
        iota = lax.broadcasted_iota(I32, (16,), 0)
        ownmask = (iota >= own0) & (iota < own0 + 8)
        init = jnp.full((16,), fill, F32)
        db = (dbuf0, dbuf1)
        vb = (vbuf0, vbuf1)
        sem = (sem0, sem1)
        base = ec * epw

        def issue(ci, nb):
            e0 = base + ci * CS
            pltpu.async_copy(d_hbm.at[pl.ds(e0, CS)], db[nb], sem[nb])
            pltpu.async_copy(h2_hbm.at[pl.ds(e0, CS)], vb[nb], sem[nb])

        issue(0, 0)

        def initrow(i, c):
            mbuf[pl.ds(i * 16, 16)] = init
            return c

        lax.fori_loop(0, msize // 16, initrow, 0)

        def pairbody(gi, c):
            for b in range(2):
                ci = gi * 2 + b
                issue(lax.rem(ci + 1, nch), 1 - b)
                pltpu.make_async_copy(
                    d_hbm.at[pl.ds(base, CS)], db[b], sem[b]).wait()
                pltpu.make_async_copy(
                    h2_hbm.at[pl.ds(base, CS)], vb[b], sem[b]).wait()

                def grp(gj, c2):
                    dvec = db[b][pl.ds(gj * 16, 16)]
                    for j in range(16):
                        dd = dvec[j]
                        s0 = dd * 8 + sbias
                        vals = vb[b][gj * 16 + j, pl.ds(off, 16)]
                        cur = mbuf[pl.ds(s0, 16)]
                        nv = (jnp.maximum(cur, vals) if is_max
                              else jnp.minimum(cur, vals))
                        mbuf[pl.ds(s0, 16)] = jnp.where(ownmask, nv, cur)
                    return c2

                lax.fori_loop(0, CS // 16, grp, 0)
            return c

        lax.fori_loop(0, nch // 2, pairbody, 0)
        pltpu.make_async_copy(d_hbm.at[pl.ds(base, CS)], db[0], sem[0]).wait()
        pltpu.make_async_copy(h2_hbm.at[pl.ds(base, CS)], vb[0], sem[0]).wait()
        pltpu.sync_copy(mbuf, out_hbm.at[ec, fg])

    return k(h2, dstp)


def _node(maxp, minp, stats, g, bb, ng, nb_, relu, e2, pad_out):
    _, n, fo = maxp.shape
    fout = HP if pad_out else fo

    def body(mx_r, mn_r, st_r, g_r, bb_r, ng_r, nb_r, out_r):
        m_hi = jnp.max(mx_r[...], axis=0)
        m_lo = jnp.min(mn_r[...], axis=0)
        s = st_r[0:1, :]
        ss = st_r[1:2, :]
        mu = s / e2
        var = ss / e2 - mu * mu
        sc = g_r[...] / jnp.sqrt(var + EPSV)
        t = bb_r[...] - mu * sc
        y = jnp.where(sc >= 0, m_hi * sc, m_lo * sc) + t
        mun = jnp.mean(y, axis=0, keepdims=True)
        varn = jnp.mean((y - mun) ** 2, axis=0, keepdims=True)
        z = (y - mun) / jnp.sqrt(varn + EPSV) * ng_r[...] + nb_r[...]
        if relu:
            z = jnp.maximum(z, 0.0)
        if pad_out:
            z = jnp.concatenate([z, jnp.zeros((n, HP - fo), F32)], axis=1)
        out_r[...] = z

    return pl.pallas_call(
        body,
        out_shape=jax.ShapeDtypeStruct((n, fout), F32),
    )(maxp, minp, stats, g.reshape(1, fo), bb.reshape(1, fo),
      ng.reshape(1, fo), nb_.reshape(1, fo))


def kernel(x, edge_index, lin_w, lin_b,
           c1_w1, c1_b1, c1_w2, c1_b2, c1_g, c1_bb, bn1_g, bn1_b,
           c2_w1, c2_b1, c2_w2, c2_b2, c2_g, c2_bb, bn2_g, bn2_b,
           c3_w1, c3_b1, c3_w2, c3_b2, c3_g, c3_bb, bn3_g, bn3_b):
    n = x.shape[0]
    o = lin_w.shape[1]
    fo = 2 * o
    e = edge_index.shape[1]
    e2 = e + n
    quant = 7680
    e2p = -(-e2 // quant) * quant
    pad = e2p - e2

    ei = edge_index.astype(I32)
    loops = jnp.arange(n, dtype=I32)
    zpad = jnp.zeros((pad,), I32)
    srcp = jnp.concatenate([ei[0], loops, zpad])
    dstp = jnp.concatenate([ei[1], loops, zpad])

    layers = [
        (c1_w1, c1_b1, c1_w2, c1_b2, c1_g, c1_bb, bn1_g, bn1_b, True),
        (c2_w1, c2_b1, c2_w2, c2_b2, c2_g, c2_bb, bn2_g, bn2_b, True),
        (c3_w1, c3_b1, c3_w2, c3_b2, c3_g, c3_bb, bn3_g, bn3_b, False),
    ]
    hp = _lin(x, lin_w, lin_b)
    fin = o
    for li, (w1, b1, w2, b2, g, bb, ng, nbb, relu) in enumerate(layers):
        m = _sc_gather(hp, srcp, dstp, fin)
        h2, stats = _edge_mlp(m, w1, b1, w2, b2, e2)
        mx = _sc_segext(h2, dstp, n, True)
        mn = lax.cond(
            jnp.any(g < 0),
            lambda hh, dd: _sc_segext(hh, dd, n, False),
            lambda hh, dd: jnp.zeros((SEC, SFG, n * 8 + 16), F32),
            h2, dstp)
        mxr = (mx[:, :, 8:8 + n * 8].reshape(SEC, SFG, n, 8)
               .transpose(0, 2, 1, 3).reshape(SEC, n, fo))
        mnr = (mn[:, :, 8:8 + n * 8].reshape(SEC, SFG, n, 8)
               .transpose(0, 2, 1, 3).reshape(SEC, n, fo))
        hp = _node(mxr, mnr, stats, g, bb, ng, nbb, relu, e2,
                   pad_out=(li < 2))
        fin = fo
    return hp


# speedup vs baseline: 2.1033x; 1.1339x over previous
import functools

import jax
import jax.numpy as jnp
from jax import lax
from jax.experimental import pallas as pl
from jax.experimental.pallas import tpu as pltpu
from jax.experimental.pallas import tpu_sc as plsc

F32 = jnp.float32
I32 = jnp.int32
EPSV = 1e-05
NCORE, NSUB = 2, 16
NW = NCORE * NSUB
CG = 120
CS = 160
SEC = 4
SFG = 8
BLK = 1280
HP = 128
NEG = -3.0e38
POS = 3.0e38


def _lin(x, lin_w, lin_b):
    n = x.shape[0]
    o = lin_w.shape[1]

    def body(x_r, lw_r, lb_r, h_r):
        h0 = jnp.dot(x_r[...], lw_r[...], preferred_element_type=F32) + lb_r[...]
        h_r[...] = jnp.concatenate([h0, jnp.zeros((n, HP - o), F32)], axis=1)

    return pl.pallas_call(
        body,
        out_shape=jax.ShapeDtypeStruct((n, HP), F32),
    )(x, lin_w, lin_b.reshape(1, o))


def _sc_gather(hp, srcp, dstp, fin):
    e2p = srcp.shape[0]
    fm = 2 * fin
    nepw = e2p // NW
    nch = nepw // CG
    mesh = plsc.VectorSubcoreMesh(core_axis_name="c", subcore_axis_name="s")

    @functools.partial(
        pl.kernel,
        out_type=jax.ShapeDtypeStruct((e2p, fm), F32),
        mesh=mesh,
        scratch_types=[
            pltpu.VMEM((CG,), I32),
            pltpu.VMEM((CG,), I32),
            pltpu.VMEM((CG,), I32),
            pltpu.VMEM((CG,), I32),
            pltpu.VMEM((CG, HP), F32),
            pltpu.VMEM((CG, HP), F32),
            pltpu.VMEM((CG, HP), F32),
            pltpu.VMEM((CG, HP), F32),
            pltpu.VMEM((CG, fm), F32),
            pltpu.VMEM((CG, fm), F32),
            pltpu.SemaphoreType.DMA,
            pltpu.SemaphoreType.DMA,
            pltpu.SemaphoreType.DMA,
            pltpu.SemaphoreType.DMA,
        ],
    )
    def k(h_hbm, s_hbm, d_hbm, out_hbm,
          dbuf0, dbuf1, sbuf0, sbuf1, dr0, dr1, sr0, sr1, ob0, ob1,
          semg0, semg1, semo0, semo1):
        wid = lax.axis_index("s") * NCORE + lax.axis_index("c")
        base = wid * nepw
        db = (dbuf0, dbuf1)
        sb = (sbuf0, sbuf1)
        dr = (dr0, dr1)
        sr = (sr0, sr1)
        ob = (ob0, ob1)
        semg = (semg0, semg1)
        semo = (semo0, semo1)

        def issue(ci, nb):
            e0 = base + ci * CG
            pltpu.sync_copy(d_hbm.at[pl.ds(e0, CG)], db[nb])
            pltpu.sync_copy(s_hbm.at[pl.ds(e0, CG)], sb[nb])
            pltpu.async_copy(h_hbm.at[db[nb]], dr[nb], semg[nb])
            pltpu.async_copy(h_hbm.at[sb[nb]], sr[nb], semg[nb])

        issue(0, 0)

        def pairbody(gi, carry):
            for b in range(2):
                ci = gi * 2 + b
                nxt = lax.rem(ci + 1, nch)
                issue(nxt, 1 - b)
                pltpu.make_async_copy(h_hbm.at[db[b]], dr[b], semg[b]).wait()
                pltpu.make_async_copy(h_hbm.at[sb[b]], sr[b], semg[b]).wait()

                @pl.when(ci >= 2)
                def _():
                    pltpu.make_async_copy(
                        ob[b], out_hbm.at[pl.ds(base, CG)], semo[b]).wait()

                def row(i, c2):
                    for j in range(fin // 16):
                        sl = pl.ds(j * 16, 16)
                        sl2 = pl.ds(fin + j * 16, 16)
                        xi = dr[b][i, sl]
                        ob[b][i, sl] = xi
                        ob[b][i, sl2] = sr[b][i, sl] - xi
                    return c2

                lax.fori_loop(0, CG, row, 0)
                e0 = base + ci * CG
                pltpu.async_copy(ob[b], out_hbm.at[pl.ds(e0, CG)], semo[b])
            return carry

        lax.fori_loop(0, nch // 2, pairbody, 0)
        pltpu.make_async_copy(h_hbm.at[db[0]], dr[0], semg[0]).wait()
        pltpu.make_async_copy(h_hbm.at[sb[0]], sr[0], semg[0]).wait()
        for b in range(2):
            pltpu.make_async_copy(
                ob[b], out_hbm.at[pl.ds(base, CG)], semo[b]).wait()

    return k(hp, srcp, dstp)


def _edge_mlp(m, w1, b1, w2, b2, e2):
    e2p, fm = m.shape
    h = w1.shape[1]
    fo = w2.shape[1]
    nb = e2p // BLK

    def body(m_r, w1_r, b1_r, w2_r, b2_r, h2_r, st_r):
        i = pl.program_id(0)
        h1 = jnp.maximum(
            jnp.dot(m_r[...], w1_r[...], preferred_element_type=F32) + b1_r[...],
            0.0)
        hh = jnp.maximum(
            jnp.dot(h1, w2_r[...], preferred_element_type=F32) + b2_r[...], 0.0)
        h2_r[...] = hh
        rows = i * BLK + lax.broadcasted_iota(I32, (BLK, 1), 0)
        hm = jnp.where(rows < e2, hh, 0.0)
        ps = jnp.sum(hm, axis=0, keepdims=True)
        pq = jnp.sum(hm * hm, axis=0, keepdims=True)

        @pl.when(i == 0)
        def _():
            st_r[...] = jnp.zeros_like(st_r)

        st_r[0:1, :] = st_r[0:1, :] + ps
        st_r[1:2, :] = st_r[1:2, :] + pq

    return pl.pallas_call(
        body,
        grid=(nb,),
        in_specs=[
            pl.BlockSpec((BLK, fm), lambda i: (i, 0)),
            pl.BlockSpec((fm, h), lambda i: (0, 0)),
            pl.BlockSpec((1, h), lambda i: (0, 0)),
            pl.BlockSpec((h, fo), lambda i: (0, 0)),
            pl.BlockSpec((1, fo), lambda i: (0, 0)),
        ],
        out_specs=[
            pl.BlockSpec((BLK, fo), lambda i: (i, 0)),
            pl.BlockSpec((8, fo), lambda i: (0, 0)),
        ],
        out_shape=(jax.ShapeDtypeStruct((e2p, fo), F32),
                   jax.ShapeDtypeStruct((8, fo), F32)),
    )(m, w1, b1.reshape(1, h), w2, b2.reshape(1, fo))


def _sc_segext(h2, dstp, n, is_max):
    e2p, fo = h2.shape
    epw = e2p // SEC
    nch = epw // CS
    nflat = n * 8
    msize = nflat + 16
    fill = NEG if is_max else POS
    mesh = plsc.VectorSubcoreMesh(core_axis_name="c", subcore_axis_name="s")

    @functools.partial(
        pl.kernel,
        out_type=jax.ShapeDtypeStruct((SEC, SFG, msize), F32),
        mesh=mesh,
        scratch_types=[
            pltpu.VMEM((msize,), F32),
            pltpu.VMEM((CS,), I32),
            pltpu.VMEM((CS,), I32),
            pltpu.VMEM((CS, 64), F32),
            pltpu.VMEM((CS, 64), F32),
            pltpu.SemaphoreType.DMA,
            pltpu.SemaphoreType.DMA,
        ],
    )
    def k(h2_hbm, d_hbm, out_hbm, mbuf, dbuf0, dbuf1, vbuf0, vbuf1,
          sem0, sem1):
        wid = lax.axis_index("s") * NCORE + lax.axis_index("c")
        ec = wid // SFG
        fg = wid - ec * SFG
        off = jnp.minimum(fg * 8, fo - 16)
        own0 = fg * 8 - off
        sbias = 8 - own0
        iota = lax.broadcasted_iota(I32, (16,), 0)
        ownmask = (iota >= own0) & (iota < own0 + 8)
        init = jnp.full((16,), fill, F32)
        db = (dbuf0, dbuf1)
        vb = (vbuf0, vbuf1)
        sem = (sem0, sem1)
        base = ec * epw

        def issue(ci, nb):
            e0 = base + ci * CS
            pltpu.async_copy(d_hbm.at[pl.ds(e0, CS)], db[nb], sem[nb])
            pltpu.async_copy(h2_hbm.at[pl.ds(e0, CS)], vb[nb], sem[nb])

        issue(0, 0)

        def initrow(i, c):
            mbuf[pl.ds(i * 16, 16)] = init
            return c

        lax.fori_loop(0, msize // 16, initrow, 0)

        def pairbody(gi, c):
            for b in range(2):
                ci = gi * 2 + b
                issue(lax.rem(ci + 1, nch), 1 - b)
                pltpu.make_async_copy(
                    d_hbm.at[pl.ds(base, CS)], db[b], sem[b]).wait()
                pltpu.make_async_copy(
                    h2_hbm.at[pl.ds(base, CS)], vb[b], sem[b]).wait()

                def grp(gj, c2):
                    dvec = db[b][pl.ds(gj * 16, 16)]
                    for j in range(16):
                        dd = dvec[j]
                        s0 = dd * 8 + sbias
                        vals = vb[b][gj * 16 + j, pl.ds(off, 16)]
                        cur = mbuf[pl.ds(s0, 16)]
                        nv = (jnp.maximum(cur, vals) if is_max
                              else jnp.minimum(cur, vals))
                        mbuf[pl.ds(s0, 16)] = jnp.where(ownmask, nv, cur)
                    return c2

                lax.fori_loop(0, CS // 16, grp, 0)
            return c

        lax.fori_loop(0, nch // 2, pairbody, 0)
        pltpu.make_async_copy(d_hbm.at[pl.ds(base, CS)], db[0], sem[0]).wait()
        pltpu.make_async_copy(h2_hbm.at[pl.ds(base, CS)], vb[0], sem[0]).wait()
        pltpu.sync_copy(mbuf, out_hbm.at[ec, fg])

    return k(h2, dstp)


def _node(maxp, minp, stats, g, bb, ng, nb_, relu, e2, pad_out):
    _, n, fo = maxp.shape
    fout = HP if pad_out else fo

    def body(mx_r, mn_r, st_r, g_r, bb_r, ng_r, nb_r, out_r):
        m_hi = jnp.max(mx_r[...], axis=0)
        m_lo = jnp.min(mn_r[...], axis=0)
        s = st_r[0:1, :]
        ss = st_r[1:2, :]
        mu = s / e2
        var = ss / e2 - mu * mu
        sc = g_r[...] / jnp.sqrt(var + EPSV)
        t = bb_r[...] - mu * sc
        y = jnp.where(sc >= 0, m_hi * sc, m_lo * sc) + t
        mun = jnp.mean(y, axis=0, keepdims=True)
        varn = jnp.mean((y - mun) ** 2, axis=0, keepdims=True)
        z = (y - mun) / jnp.sqrt(varn + EPSV) * ng_r[...] + nb_r[...]
        if relu:
            z = jnp.maximum(z, 0.0)
        if pad_out:
            z = jnp.concatenate([z, jnp.zeros((n, HP - fo), F32)], axis=1)
        out_r[...] = z

    return pl.pallas_call(
        body,
        out_shape=jax.ShapeDtypeStruct((n, fout), F32),
    )(maxp, minp, stats, g.reshape(1, fo), bb.reshape(1, fo),
      ng.reshape(1, fo), nb_.reshape(1, fo))


def kernel(x, edge_index, lin_w, lin_b,
           c1_w1, c1_b1, c1_w2, c1_b2, c1_g, c1_bb, bn1_g, bn1_b,
           c2_w1, c2_b1, c2_w2, c2_b2, c2_g, c2_bb, bn2_g, bn2_b,
           c3_w1, c3_b1, c3_w2, c3_b2, c3_g, c3_bb, bn3_g, bn3_b):
    n = x.shape[0]
    o = lin_w.shape[1]
    fo = 2 * o
    e = edge_index.shape[1]
    e2 = e + n
    quant = 7680
    e2p = -(-e2 // quant) * quant
    pad = e2p - e2

    ei = edge_index.astype(I32)
    loops = jnp.arange(n, dtype=I32)
    zpad = jnp.zeros((pad,), I32)
    srcp = jnp.concatenate([ei[0], loops, zpad])
    dstp = jnp.concatenate([ei[1], loops, zpad])

    layers = [
        (c1_w1, c1_b1, c1_w2, c1_b2, c1_g, c1_bb, bn1_g, bn1_b, True),
        (c2_w1, c2_b1, c2_w2, c2_b2, c2_g, c2_bb, bn2_g, bn2_b, True),
        (c3_w1, c3_b1, c3_w2, c3_b2, c3_g, c3_bb, bn3_g, bn3_b, False),
    ]
    hp = _lin(x, lin_w, lin_b)
    fin = o
    for li, (w1, b1, w2, b2, g, bb, ng, nbb, relu) in enumerate(layers):
        m = _sc_gather(hp, srcp, dstp, fin)
        h2, stats = _edge_mlp(m, w1, b1, w2, b2, e2)
        mx = _sc_segext(h2, dstp, n, True)
        mn = lax.cond(
            jnp.any(g < 0),
            lambda hh, dd: _sc_segext(hh, dd, n, False),
            lambda hh, dd: jnp.zeros((SEC, SFG, n * 8 + 16), F32),
            h2, dstp)
        mxr = (mx[:, :, 8:8 + n * 8].reshape(SEC, SFG, n, 8)
               .transpose(0, 2, 1, 3).reshape(SEC, n, fo))
        mnr = (mn[:, :, 8:8 + n * 8].reshape(SEC, SFG, n, 8)
               .transpose(0, 2, 1, 3).reshape(SEC, n, fo))
        hp = _node(mxr, mnr, stats, g, bb, ng, nbb, relu, e2,
                   pad_out=(li < 2))
        fin = fo
    return hp


# speedup vs baseline: 2.2029x; 1.0474x over previous
import functools

import jax
import jax.numpy as jnp
from jax import lax
from jax.experimental import pallas as pl
from jax.experimental.pallas import tpu as pltpu
from jax.experimental.pallas import tpu_sc as plsc

F32 = jnp.float32
I32 = jnp.int32
EPSV = 1e-05
NCORE, NSUB = 2, 16
NW = NCORE * NSUB
CG = 120
CS = 160
SEC = 4
SFG = 8
BLK = 2560
HP = 128
NEG = -3.0e38
POS = 3.0e38


def _lin(x, lin_w, lin_b):
    n = x.shape[0]
    o = lin_w.shape[1]

    def body(x_r, lw_r, lb_r, h_r):
        h0 = jnp.dot(x_r[...], lw_r[...], preferred_element_type=F32) + lb_r[...]
        h_r[...] = jnp.concatenate([h0, jnp.zeros((n, HP - o), F32)], axis=1)

    return pl.pallas_call(
        body,
        out_shape=jax.ShapeDtypeStruct((n, HP), F32),
    )(x, lin_w, lin_b.reshape(1, o))


def _sc_gather(hp, srcp, dstp, fin):
    e2p = srcp.shape[0]
    fm = 2 * fin
    nepw = e2p // NW
    nch = nepw // CG
    mesh = plsc.VectorSubcoreMesh(core_axis_name="c", subcore_axis_name="s")

    @functools.partial(
        pl.kernel,
        out_type=jax.ShapeDtypeStruct((e2p, fm), F32),
        mesh=mesh,
        scratch_types=[
            pltpu.VMEM((CG,), I32),
            pltpu.VMEM((CG,), I32),
            pltpu.VMEM((CG,), I32),
            pltpu.VMEM((CG,), I32),
            pltpu.VMEM((CG, HP), F32),
            pltpu.VMEM((CG, HP), F32),
            pltpu.VMEM((CG, HP), F32),
            pltpu.VMEM((CG, HP), F32),
            pltpu.VMEM((CG, fm), F32),
            pltpu.VMEM((CG, fm), F32),
            pltpu.SemaphoreType.DMA,
            pltpu.SemaphoreType.DMA,
            pltpu.SemaphoreType.DMA,
            pltpu.SemaphoreType.DMA,
        ],
    )
    def k(h_hbm, s_hbm, d_hbm, out_hbm,
          dbuf0, dbuf1, sbuf0, sbuf1, dr0, dr1, sr0, sr1, ob0, ob1,
          semg0, semg1, semo0, semo1):
        wid = lax.axis_index("s") * NCORE + lax.axis_index("c")
        base = wid * nepw
        db = (dbuf0, dbuf1)
        sb = (sbuf0, sbuf1)
        dr = (dr0, dr1)
        sr = (sr0, sr1)
        ob = (ob0, ob1)
        semg = (semg0, semg1)
        semo = (semo0, semo1)

        def issue(ci, nb):
            e0 = base + ci * CG
            pltpu.sync_copy(d_hbm.at[pl.ds(e0, CG)], db[nb])
            pltpu.sync_copy(s_hbm.at[pl.ds(e0, CG)], sb[nb])
            pltpu.async_copy(h_hbm.at[db[nb]], dr[nb], semg[nb])
            pltpu.async_copy(h_hbm.at[sb[nb]], sr[nb], semg[nb])

        issue(0, 0)

        def pairbody(gi, carry):
            for b in range(2):
                ci = gi * 2 + b
                nxt = lax.rem(ci + 1, nch)
                issue(nxt, 1 - b)
                pltpu.make_async_copy(h_hbm.at[db[b]], dr[b], semg[b]).wait()
                pltpu.make_async_copy(h_hbm.at[sb[b]], sr[b], semg[b]).wait()

                @pl.when(ci >= 2)
                def _():
                    pltpu.make_async_copy(
                        ob[b], out_hbm.at[pl.ds(base, CG)], semo[b]).wait()

                def row(i, c2):
                    for j in range(fin // 16):
                        sl = pl.ds(j * 16, 16)
                        sl2 = pl.ds(fin + j * 16, 16)
                        xi = dr[b][i, sl]
                        ob[b][i, sl] = xi
                        ob[b][i, sl2] = sr[b][i, sl] - xi
                    return c2

                lax.fori_loop(0, CG, row, 0)
                e0 = base + ci * CG
                pltpu.async_copy(ob[b], out_hbm.at[pl.ds(e0, CG)], semo[b])
            return carry

        lax.fori_loop(0, nch // 2, pairbody, 0)
        pltpu.make_async_copy(h_hbm.at[db[0]], dr[0], semg[0]).wait()
        pltpu.make_async_copy(h_hbm.at[sb[0]], sr[0], semg[0]).wait()
        for b in range(2):
            pltpu.make_async_copy(
                ob[b], out_hbm.at[pl.ds(base, CG)], semo[b]).wait()

    return k(hp, srcp, dstp)


def _edge_mlp(m, w1, b1, w2, b2, e2):
    e2p, fm = m.shape
    h = w1.shape[1]
    fo = w2.shape[1]
    nb = e2p // BLK

    def body(m_r, w1_r, b1_r, w2_r, b2_r, h2_r, st_r):
        i = pl.program_id(0)
        h1 = jnp.maximum(
            jnp.dot(m_r[...], w1_r[...], preferred_element_type=F32) + b1_r[...],
            0.0)
        hh = jnp.maximum(
            jnp.dot(h1, w2_r[...], preferred_element_type=F32) + b2_r[...], 0.0)
        h2_r[...] = hh
        rows = i * BLK + lax.broadcasted_iota(I32, (BLK, 1), 0)
        hm = jnp.where(rows < e2, hh, 0.0)
        ps = jnp.sum(hm, axis=0, keepdims=True)
        pq = jnp.sum(hm * hm, axis=0, keepdims=True)

        @pl.when(i == 0)
        def _():
            st_r[...] = jnp.zeros_like(st_r)

        st_r[0:1, :] = st_r[0:1, :] + ps
        st_r[1:2, :] = st_r[1:2, :] + pq

    return pl.pallas_call(
        body,
        grid=(nb,),
        in_specs=[
            pl.BlockSpec((BLK, fm), lambda i: (i, 0)),
            pl.BlockSpec((fm, h), lambda i: (0, 0)),
            pl.BlockSpec((1, h), lambda i: (0, 0)),
            pl.BlockSpec((h, fo), lambda i: (0, 0)),
            pl.BlockSpec((1, fo), lambda i: (0, 0)),
        ],
        out_specs=[
            pl.BlockSpec((BLK, fo), lambda i: (i, 0)),
            pl.BlockSpec((8, fo), lambda i: (0, 0)),
        ],
        out_shape=(jax.ShapeDtypeStruct((e2p, fo), F32),
                   jax.ShapeDtypeStruct((8, fo), F32)),
    )(m, w1, b1.reshape(1, h), w2, b2.reshape(1, fo))


def _sc_segext(h2, dstp, n, is_max):
    e2p, fo = h2.shape
    epw = e2p // SEC
    nch = epw // CS
    nflat = n * 8
    msize = nflat + 16
    fill = NEG if is_max else POS
    mesh = plsc.VectorSubcoreMesh(core_axis_name="c", subcore_axis_name="s")

    @functools.partial(
        pl.kernel,
        out_type=jax.ShapeDtypeStruct((SEC, SFG, msize), F32),
        mesh=mesh,
        scratch_types=[
            pltpu.VMEM((msize,), F32),
            pltpu.VMEM((CS,), I32),
            pltpu.VMEM((CS,), I32),
            pltpu.VMEM((CS, 64), F32),
            pltpu.VMEM((CS, 64), F32),
            pltpu.SemaphoreType.DMA,
            pltpu.SemaphoreType.DMA,
        ],
    )
    def k(h2_hbm, d_hbm, out_hbm, mbuf, dbuf0, dbuf1, vbuf0, vbuf1,
          sem0, sem1):
        wid = lax.axis_index("s") * NCORE + lax.axis_index("c")
        ec = wid // SFG
        fg = wid - ec * SFG
        off = jnp.minimum(fg * 8, fo - 16)
        own0 = fg * 8 - off
        sbias = 8 - own0
        iota = lax.broadcasted_iota(I32, (16,), 0)
        ownmask = (iota >= own0) & (iota < own0 + 8)
        init = jnp.full((16,), fill, F32)
        db = (dbuf0, dbuf1)
        vb = (vbuf0, vbuf1)
        sem = (sem0, sem1)
        base = ec * epw

        def issue(ci, nb):
            e0 = base + ci * CS
            pltpu.async_copy(d_hbm.at[pl.ds(e0, CS)], db[nb], sem[nb])
            pltpu.async_copy(h2_hbm.at[pl.ds(e0, CS)], vb[nb], sem[nb])

        issue(0, 0)

        def initrow(i, c):
            mbuf[pl.ds(i * 16, 16)] = init
            return c

        lax.fori_loop(0, msize // 16, initrow, 0)

        def pairbody(gi, c):
            for b in range(2):
                ci = gi * 2 + b
                issue(lax.rem(ci + 1, nch), 1 - b)
                pltpu.make_async_copy(
                    d_hbm.at[pl.ds(base, CS)], db[b], sem[b]).wait()
                pltpu.make_async_copy(
                    h2_hbm.at[pl.ds(base, CS)], vb[b], sem[b]).wait()

                def grp(gj, c2):
                    dvec = db[b][pl.ds(gj * 16, 16)]
                    for j in range(16):
                        dd = dvec[j]
                        s0 = dd * 8 + sbias
                        vals = vb[b][gj * 16 + j, pl.ds(off, 16)]
                        cur = mbuf[pl.ds(s0, 16)]
                        nv = (jnp.maximum(cur, vals) if is_max
                              else jnp.minimum(cur, vals))
                        mbuf[pl.ds(s0, 16)] = jnp.where(ownmask, nv, cur)
                    return c2

                lax.fori_loop(0, CS // 16, grp, 0)
            return c

        lax.fori_loop(0, nch // 2, pairbody, 0)
        pltpu.make_async_copy(d_hbm.at[pl.ds(base, CS)], db[0], sem[0]).wait()
        pltpu.make_async_copy(h2_hbm.at[pl.ds(base, CS)], vb[0], sem[0]).wait()
        pltpu.sync_copy(mbuf, out_hbm.at[ec, fg])

    return k(h2, dstp)


def _node(maxp, minp, stats, g, bb, ng, nb_, relu, e2, pad_out):
    _, n, fo = maxp.shape
    fout = HP if pad_out else fo

    def body(mx_r, mn_r, st_r, g_r, bb_r, ng_r, nb_r, out_r):
        m_hi = jnp.max(mx_r[...], axis=0)
        m_lo = jnp.min(mn_r[...], axis=0)
        s = st_r[0:1, :]
        ss = st_r[1:2, :]
        mu = s / e2
        var = ss / e2 - mu * mu
        sc = g_r[...] / jnp.sqrt(var + EPSV)
        t = bb_r[...] - mu * sc
        y = jnp.where(sc >= 0, m_hi * sc, m_lo * sc) + t
        mun = jnp.mean(y, axis=0, keepdims=True)
        varn = jnp.mean((y - mun) ** 2, axis=0, keepdims=True)
        z = (y - mun) / jnp.sqrt(varn + EPSV) * ng_r[...] + nb_r[...]
        if relu:
            z = jnp.maximum(z, 0.0)
        if pad_out:
            z = jnp.concatenate([z, jnp.zeros((n, HP - fo), F32)], axis=1)
        out_r[...] = z

    return pl.pallas_call(
        body,
        out_shape=jax.ShapeDtypeStruct((n, fout), F32),
    )(maxp, minp, stats, g.reshape(1, fo), bb.reshape(1, fo),
      ng.reshape(1, fo), nb_.reshape(1, fo))


def kernel(x, edge_index, lin_w, lin_b,
           c1_w1, c1_b1, c1_w2, c1_b2, c1_g, c1_bb, bn1_g, bn1_b,
           c2_w1, c2_b1, c2_w2, c2_b2, c2_g, c2_bb, bn2_g, bn2_b,
           c3_w1, c3_b1, c3_w2, c3_b2, c3_g, c3_bb, bn3_g, bn3_b):
    n = x.shape[0]
    o = lin_w.shape[1]
    fo = 2 * o
    e = edge_index.shape[1]
    e2 = e + n
    quant = 7680
    e2p = -(-e2 // quant) * quant
    pad = e2p - e2

    ei = edge_index.astype(I32)
    loops = jnp.arange(n, dtype=I32)
    zpad = jnp.zeros((pad,), I32)
    srcp = jnp.concatenate([ei[0], loops, zpad])
    dstp = jnp.concatenate([ei[1], loops, zpad])

    layers = [
        (c1_w1, c1_b1, c1_w2, c1_b2, c1_g, c1_bb, bn1_g, bn1_b, True),
        (c2_w1, c2_b1, c2_w2, c2_b2, c2_g, c2_bb, bn2_g, bn2_b, True),
        (c3_w1, c3_b1, c3_w2, c3_b2, c3_g, c3_bb, bn3_g, bn3_b, False),
    ]
    hp = _lin(x, lin_w, lin_b)
    fin = o
    for li, (w1, b1, w2, b2, g, bb, ng, nbb, relu) in enumerate(layers):
        m = _sc_gather(hp, srcp, dstp, fin)
        h2, stats = _edge_mlp(m, w1, b1, w2, b2, e2)
        mx = _sc_segext(h2, dstp, n, True)
        mn = lax.cond(
            jnp.any(g < 0),
            lambda hh, dd: _sc_segext(hh, dd, n, False),
            lambda hh, dd: jnp.zeros((SEC, SFG, n * 8 + 16), F32),
            h2, dstp)
        mxr = (mx[:, :, 8:8 + n * 8].reshape(SEC, SFG, n, 8)
               .transpose(0, 2, 1, 3).reshape(SEC, n, fo))
        mnr = (mn[:, :, 8:8 + n * 8].reshape(SEC, SFG, n, 8)
               .transpose(0, 2, 1, 3).reshape(SEC, n, fo))
        hp = _node(mxr, mnr, stats, g, bb, ng, nbb, relu, e2,
                   pad_out=(li < 2))
        fin = fo
    return hp
